# baseline (speedup 1.0000x reference)
import math

import jax
import jax.numpy as jnp
from jax.experimental import pallas as pl
from jax.experimental.pallas import tpu as pltpu

_B, _L = 8, 512
_NB = 4
_G = _B // _NB
_D_IN = 24
_D_EU = 2
_EMB = 64
_CONV = 64
_SCALE = 1.0 / math.sqrt(_EMB)
_PREC = jax.lax.Precision.DEFAULT


def _mm(x, y):
    return jax.lax.dot_general(x, y, (((1,), (0,)), ((), ())),
                               precision=_PREC, preferred_element_type=jnp.float32)


def _mm_t(x, y):
    return jax.lax.dot_general(x, y, (((1,), (1,)), ((), ())),
                               precision=_PREC, preferred_element_type=jnp.float32)


def _fused_kernel(lens_ref, inputs_ref, coords_ref, cc_ref,
                  W_c_ref, b_c_ref, W_cc_ref, b_cc_ref,
                  W_a_ref, b_a_ref, W_b_ref, b_b_ref,
                  W_l1_ref, W_r1_ref, b1_ref, W_l2_ref, W_r2_ref, b2_ref,
                  head_ref, diff_ref):
    g = pl.program_id(0)

    x_in3 = inputs_ref[...]
    ctr3 = x_in3 - x_in3[:, 0:1, :]
    x_in = x_in3.reshape(_NB * _L, _D_IN)
    ctr = ctr3.reshape(_NB * _L, _D_IN)
    co = coords_ref[...].reshape(_NB * _L, _D_EU)
    ccv = cc_ref[...].reshape(_NB * _L, _D_EU)

    Wc = W_c_ref[...]
    Wcc = W_cc_ref[...]
    i_emb = _mm(x_in, Wc[:_D_IN]) + _mm(co, Wc[_D_IN:]) + b_c_ref[...]
    ic_emb = _mm(ctr, Wcc[:_D_IN]) + _mm(ccv, Wcc[_D_IN:]) + b_cc_ref[...]

    a2 = _mm(i_emb, W_a_ref[...]) + b_a_ref[...]
    bt2 = _mm(i_emb, W_b_ref[...]) + b_b_ref[...]
    a_c2 = _mm(ic_emb, W_a_ref[...]) + b_a_ref[...]
    bt_c2 = _mm(ic_emb, W_b_ref[...]) + b_b_ref[...]
    x2 = i_emb - ic_emb

    col = jax.lax.broadcasted_iota(jnp.int32, (_L, _L), 1)
    row = jax.lax.broadcasted_iota(jnp.int32, (_L, 1), 0)

    heads = []
    diffs = []
    for j in range(_NB):
        sl = slice(j * _L, (j + 1) * _L)
        n = lens_ref[g * _NB + j]
        a, bt, a_c, bt_c, x = a2[sl], bt2[sl], a_c2[sl], bt_c2[sl], x2[sl]

        adj = _mm_t(a, bt) * _SCALE
        adj_c = _mm_t(a_c, bt_c) * _SCALE

        validc = col < n
        d = jnp.where(validc, adj - adj_c, 0.0)
        diffs.append(jnp.sum(d * d) * (1.0 / (_B * _L * _L)))

        big = jnp.where(validc, adj, jnp.inf)
        minv = jnp.min(big, axis=1, keepdims=True)
        excl = jnp.max(jnp.where(big == minv, col, -1), axis=1)
        onehot = (col == excl[:, None]).astype(jnp.float32)

        x_excl = _mm(onehot, x)
        rowmask = row < n
        inv_deg = 1.0 / jnp.maximum(n - 1, 1).astype(jnp.float32)
        s_x = jnp.sum(jnp.where(rowmask, x, 0.0), axis=0, keepdims=True)
        aggr1 = (s_x - x_excl) * inv_deg
        h = jax.nn.relu(_mm(aggr1, W_l1_ref[...]) + _mm(x, W_r1_ref[...])
                        + b1_ref[...])

        s_h = jnp.sum(jnp.where(rowmask, h, 0.0), axis=0, keepdims=True)
        h_excl0 = _mm(onehot[0:1, :], h)
        aggr2 = (s_h - h_excl0) * inv_deg
        out0 = (_mm(aggr2, W_l2_ref[...]) + _mm(h[0:1, :], W_r2_ref[...])
                + b2_ref[...])
        heads.append(out0.reshape(1, 1, 1))

    head_ref[...] = jnp.concatenate(heads, axis=0)
    diff_ref[...] = jnp.stack(diffs).reshape(_NB, 1, 1)


def kernel(inputs, coords, cc, targets, input_lenths, W_c, b_c, W_cc, b_cc,
           W_a, b_a, W_b, b_b, W_l1, W_r1, b1, W_l2, W_r2, b2):
    lens = input_lenths.astype(jnp.int32)
    b_c2 = b_c.reshape(1, _EMB)
    b_cc2 = b_cc.reshape(1, _EMB)
    b_a2 = b_a.reshape(1, _EMB)
    b_b2 = b_b.reshape(1, _EMB)
    b12 = b1.reshape(1, _CONV)
    b22 = b2.reshape(1, 1)

    def full(arr):
        return pl.BlockSpec(arr.shape, lambda g, s: (0,) * arr.ndim)

    grid_spec = pltpu.PrefetchScalarGridSpec(
        num_scalar_prefetch=1,
        grid=(_G,),
        in_specs=[
            pl.BlockSpec((_NB, _L, _D_IN), lambda g, s: (g, 0, 0)),
            pl.BlockSpec((_NB, _L, _D_EU), lambda g, s: (g, 0, 0)),
            pl.BlockSpec((_NB, _L, _D_EU), lambda g, s: (g, 0, 0)),
            full(W_c), full(b_c2), full(W_cc), full(b_cc2),
            full(W_a), full(b_a2), full(W_b), full(b_b2),
            full(W_l1), full(W_r1), full(b12), full(W_l2), full(W_r2), full(b22),
        ],
        out_specs=[
            pl.BlockSpec((_NB, 1, 1), lambda g, s: (g, 0, 0)),
            pl.BlockSpec((_NB, 1, 1), lambda g, s: (g, 0, 0)),
        ],
    )
    head, diff = pl.pallas_call(
        _fused_kernel,
        grid_spec=grid_spec,
        out_shape=[jax.ShapeDtypeStruct((_B, 1, 1), jnp.float32),
                   jax.ShapeDtypeStruct((_B, 1, 1), jnp.float32)],
        compiler_params=pltpu.CompilerParams(dimension_semantics=("parallel",)),
    )(lens, inputs, coords, cc, W_c, b_c2, W_cc, b_cc2,
      W_a, b_a2, W_b, b_b2, W_l1, W_r1, b12, W_l2, W_r2, b22)

    output_head = head.reshape(_B, 1)
    target_head = targets[:, 0, :]
    return output_head, jnp.sum(diff).reshape(()), target_head


# speedup vs baseline: 1.0057x; 1.0057x over previous
import math

import jax
import jax.numpy as jnp
from jax.experimental import pallas as pl
from jax.experimental.pallas import tpu as pltpu

_B, _L = 8, 512
_NB = 2
_G = _B // _NB
_D_IN = 24
_D_EU = 2
_EMB = 64
_CONV = 64
_SCALE = 1.0 / math.sqrt(_EMB)
_PREC = jax.lax.Precision.DEFAULT


def _mm(x, y):
    return jax.lax.dot_general(x, y, (((1,), (0,)), ((), ())),
                               precision=_PREC, preferred_element_type=jnp.float32)


def _mm_t(x, y):
    return jax.lax.dot_general(x, y, (((1,), (1,)), ((), ())),
                               precision=_PREC, preferred_element_type=jnp.float32)


def _fused_kernel(lens_ref, inputs_ref, coords_ref, cc_ref,
                  W_c_ref, b_c_ref, W_cc_ref, b_cc_ref,
                  W_a_ref, b_a_ref, W_b_ref, b_b_ref,
                  W_l1_ref, W_r1_ref, b1_ref, W_l2_ref, W_r2_ref, b2_ref,
                  head_ref, diff_ref):
    g = pl.program_id(0)

    x_in3 = inputs_ref[...]
    ctr3 = x_in3 - x_in3[:, 0:1, :]
    x_in = x_in3.reshape(_NB * _L, _D_IN)
    ctr = ctr3.reshape(_NB * _L, _D_IN)
    co = coords_ref[...].reshape(_NB * _L, _D_EU)
    ccv = cc_ref[...].reshape(_NB * _L, _D_EU)

    Wc = W_c_ref[...]
    Wcc = W_cc_ref[...]
    i_emb = _mm(x_in, Wc[:_D_IN]) + _mm(co, Wc[_D_IN:]) + b_c_ref[...]
    ic_emb = _mm(ctr, Wcc[:_D_IN]) + _mm(ccv, Wcc[_D_IN:]) + b_cc_ref[...]

    a2 = _mm(i_emb, W_a_ref[...]) + b_a_ref[...]
    bt2 = _mm(i_emb, W_b_ref[...]) + b_b_ref[...]
    a_c2 = _mm(ic_emb, W_a_ref[...]) + b_a_ref[...]
    bt_c2 = _mm(ic_emb, W_b_ref[...]) + b_b_ref[...]
    x2 = i_emb - ic_emb

    col = jax.lax.broadcasted_iota(jnp.int32, (_L, _L), 1)
    row = jax.lax.broadcasted_iota(jnp.int32, (_L, 1), 0)

    heads = []
    diffs = []
    for j in range(_NB):
        sl = slice(j * _L, (j + 1) * _L)
        n = lens_ref[g * _NB + j]
        a, bt, a_c, bt_c, x = a2[sl], bt2[sl], a_c2[sl], bt_c2[sl], x2[sl]

        adj = _mm_t(a, bt) * _SCALE
        adj_c = _mm_t(a_c, bt_c) * _SCALE

        validc = col < n
        d = jnp.where(validc, adj - adj_c, 0.0)
        diffs.append(jnp.sum(d * d) * (1.0 / (_B * _L * _L)))

        big = jnp.where(validc, adj, jnp.inf)
        minv = jnp.min(big, axis=1, keepdims=True)
        excl = jnp.max(jnp.where(big == minv, col, -1), axis=1)
        onehot = (col == excl[:, None]).astype(jnp.bfloat16)

        x_excl = _mm(onehot, x.astype(jnp.bfloat16))
        rowmask = row < n
        inv_deg = 1.0 / jnp.maximum(n - 1, 1).astype(jnp.float32)
        s_x = jnp.sum(jnp.where(rowmask, x, 0.0), axis=0, keepdims=True)
        aggr1 = (s_x - x_excl) * inv_deg
        h = jax.nn.relu(_mm(aggr1, W_l1_ref[...]) + _mm(x, W_r1_ref[...])
                        + b1_ref[...])

        s_h = jnp.sum(jnp.where(rowmask, h, 0.0), axis=0, keepdims=True)
        h_excl0 = _mm(onehot[0:1, :], h.astype(jnp.bfloat16))
        aggr2 = (s_h - h_excl0) * inv_deg
        out0 = (_mm(aggr2, W_l2_ref[...]) + _mm(h[0:1, :], W_r2_ref[...])
                + b2_ref[...])
        heads.append(out0.reshape(1, 1, 1))

    head_ref[...] = jnp.concatenate(heads, axis=0)
    diff_ref[...] = jnp.stack(diffs).reshape(_NB, 1, 1)


def kernel(inputs, coords, cc, targets, input_lenths, W_c, b_c, W_cc, b_cc,
           W_a, b_a, W_b, b_b, W_l1, W_r1, b1, W_l2, W_r2, b2):
    lens = input_lenths.astype(jnp.int32)
    b_c2 = b_c.reshape(1, _EMB)
    b_cc2 = b_cc.reshape(1, _EMB)
    b_a2 = b_a.reshape(1, _EMB)
    b_b2 = b_b.reshape(1, _EMB)
    b12 = b1.reshape(1, _CONV)
    b22 = b2.reshape(1, 1)

    def full(arr):
        return pl.BlockSpec(arr.shape, lambda g, s: (0,) * arr.ndim)

    grid_spec = pltpu.PrefetchScalarGridSpec(
        num_scalar_prefetch=1,
        grid=(_G,),
        in_specs=[
            pl.BlockSpec((_NB, _L, _D_IN), lambda g, s: (g, 0, 0)),
            pl.BlockSpec((_NB, _L, _D_EU), lambda g, s: (g, 0, 0)),
            pl.BlockSpec((_NB, _L, _D_EU), lambda g, s: (g, 0, 0)),
            full(W_c), full(b_c2), full(W_cc), full(b_cc2),
            full(W_a), full(b_a2), full(W_b), full(b_b2),
            full(W_l1), full(W_r1), full(b12), full(W_l2), full(W_r2), full(b22),
        ],
        out_specs=[
            pl.BlockSpec((_NB, 1, 1), lambda g, s: (g, 0, 0)),
            pl.BlockSpec((_NB, 1, 1), lambda g, s: (g, 0, 0)),
        ],
    )
    head, diff = pl.pallas_call(
        _fused_kernel,
        grid_spec=grid_spec,
        out_shape=[jax.ShapeDtypeStruct((_B, 1, 1), jnp.float32),
                   jax.ShapeDtypeStruct((_B, 1, 1), jnp.float32)],
        compiler_params=pltpu.CompilerParams(dimension_semantics=("parallel",)),
    )(lens, inputs, coords, cc, W_c, b_c2, W_cc, b_cc2,
      W_a, b_a2, W_b, b_b2, W_l1, W_r1, b12, W_l2, W_r2, b22)

    output_head = head.reshape(_B, 1)
    target_head = targets[:, 0, :]
    return output_head, jnp.sum(diff).reshape(()), target_head


# speedup vs baseline: 1.0137x; 1.0079x over previous
import math

import jax
import jax.numpy as jnp
from jax.experimental import pallas as pl
from jax.experimental.pallas import tpu as pltpu

_B, _L = 8, 512
_NB = 2
_G = _B // _NB
_D_IN = 24
_D_EU = 2
_EMB = 64
_CONV = 64
_SCALE = 1.0 / math.sqrt(_EMB)
_PREC = jax.lax.Precision.DEFAULT


def _mm(x, y):
    return jax.lax.dot_general(x, y, (((1,), (0,)), ((), ())),
                               precision=_PREC, preferred_element_type=jnp.float32)


def _mm_t(x, y):
    return jax.lax.dot_general(x, y, (((1,), (1,)), ((), ())),
                               precision=_PREC, preferred_element_type=jnp.float32)


def _fused_kernel(lens_ref, inputs_ref, coords_ref, cc_ref,
                  W_c_ref, b_c_ref, W_cc_ref, b_cc_ref,
                  W_a_ref, b_a_ref, W_b_ref, b_b_ref,
                  W_l1_ref, W_r1_ref, b1_ref, W_l2_ref, W_r2_ref, b2_ref,
                  head_ref, diff_ref):
    g = pl.program_id(0)

    x_in3 = inputs_ref[...]
    ctr3 = x_in3 - x_in3[:, 0:1, :]
    x_in = x_in3.reshape(_NB * _L, _D_IN)
    ctr = ctr3.reshape(_NB * _L, _D_IN)
    co = coords_ref[...].reshape(_NB * _L, _D_EU)
    ccv = cc_ref[...].reshape(_NB * _L, _D_EU)

    Wc = W_c_ref[...]
    Wcc = W_cc_ref[...]
    i_emb = _mm(x_in, Wc[:_D_IN]) + _mm(co, Wc[_D_IN:]) + b_c_ref[...]
    ic_emb = _mm(ctr, Wcc[:_D_IN]) + _mm(ccv, Wcc[_D_IN:]) + b_cc_ref[...]

    a2 = _mm(i_emb, W_a_ref[...]) + b_a_ref[...]
    bt2 = _mm(i_emb, W_b_ref[...]) + b_b_ref[...]
    a_c2 = _mm(ic_emb, W_a_ref[...]) + b_a_ref[...]
    bt_c2 = _mm(ic_emb, W_b_ref[...]) + b_b_ref[...]
    x2 = i_emb - ic_emb

    col = jax.lax.broadcasted_iota(jnp.int32, (_L, _L), 1)
    row = jax.lax.broadcasted_iota(jnp.int32, (_L, 1), 0)
    col1 = jax.lax.broadcasted_iota(jnp.int32, (1, _L), 1)

    heads = []
    diffs = []
    for j in range(_NB):
        sl = slice(j * _L, (j + 1) * _L)
        n = lens_ref[g * _NB + j]
        a, bt, a_c, bt_c, x = a2[sl], bt2[sl], a_c2[sl], bt_c2[sl], x2[sl]
        rowmask = row < n

        btm = jnp.where(rowmask, bt, 0.0)
        bt_cm = jnp.where(rowmask, bt_c, 0.0)
        adj = _mm_t(a, btm) * _SCALE
        adj_c = _mm_t(a_c, bt_cm) * _SCALE

        s = adj - adj_c
        diffs.append(jnp.sum(s * s) * (1.0 / (_B * _L * _L)))

        pen = jnp.where(col1 < n, 0.0, jnp.inf)
        big = adj + pen
        minv = jnp.min(big, axis=1, keepdims=True)
        excl = jnp.max(jnp.where(big == minv, col, -1), axis=1)
        onehot = (col == excl[:, None]).astype(jnp.bfloat16)

        x_excl = _mm(onehot, x.astype(jnp.bfloat16))
        inv_deg = 1.0 / jnp.maximum(n - 1, 1).astype(jnp.float32)
        s_x = jnp.sum(jnp.where(rowmask, x, 0.0), axis=0, keepdims=True)
        aggr1 = (s_x - x_excl) * inv_deg
        h = jax.nn.relu(_mm(aggr1, W_l1_ref[...]) + _mm(x, W_r1_ref[...])
                        + b1_ref[...])

        s_h = jnp.sum(jnp.where(rowmask, h, 0.0), axis=0, keepdims=True)
        h_excl0 = _mm(onehot[0:1, :], h.astype(jnp.bfloat16))
        aggr2 = (s_h - h_excl0) * inv_deg
        out0 = (_mm(aggr2, W_l2_ref[...]) + _mm(h[0:1, :], W_r2_ref[...])
                + b2_ref[...])
        heads.append(out0.reshape(1, 1, 1))

    head_ref[...] = jnp.concatenate(heads, axis=0)
    diff_ref[...] = jnp.stack(diffs).reshape(_NB, 1, 1)


def kernel(inputs, coords, cc, targets, input_lenths, W_c, b_c, W_cc, b_cc,
           W_a, b_a, W_b, b_b, W_l1, W_r1, b1, W_l2, W_r2, b2):
    lens = input_lenths.astype(jnp.int32)
    b_c2 = b_c.reshape(1, _EMB)
    b_cc2 = b_cc.reshape(1, _EMB)
    b_a2 = b_a.reshape(1, _EMB)
    b_b2 = b_b.reshape(1, _EMB)
    b12 = b1.reshape(1, _CONV)
    b22 = b2.reshape(1, 1)

    def full(arr):
        return pl.BlockSpec(arr.shape, lambda g, s: (0,) * arr.ndim)

    grid_spec = pltpu.PrefetchScalarGridSpec(
        num_scalar_prefetch=1,
        grid=(_G,),
        in_specs=[
            pl.BlockSpec((_NB, _L, _D_IN), lambda g, s: (g, 0, 0)),
            pl.BlockSpec((_NB, _L, _D_EU), lambda g, s: (g, 0, 0)),
            pl.BlockSpec((_NB, _L, _D_EU), lambda g, s: (g, 0, 0)),
            full(W_c), full(b_c2), full(W_cc), full(b_cc2),
            full(W_a), full(b_a2), full(W_b), full(b_b2),
            full(W_l1), full(W_r1), full(b12), full(W_l2), full(W_r2), full(b22),
        ],
        out_specs=[
            pl.BlockSpec((_NB, 1, 1), lambda g, s: (g, 0, 0)),
            pl.BlockSpec((_NB, 1, 1), lambda g, s: (g, 0, 0)),
        ],
    )
    head, diff = pl.pallas_call(
        _fused_kernel,
        grid_spec=grid_spec,
        out_shape=[jax.ShapeDtypeStruct((_B, 1, 1), jnp.float32),
                   jax.ShapeDtypeStruct((_B, 1, 1), jnp.float32)],
        compiler_params=pltpu.CompilerParams(dimension_semantics=("parallel",)),
    )(lens, inputs, coords, cc, W_c, b_c2, W_cc, b_cc2,
      W_a, b_a2, W_b, b_b2, W_l1, W_r1, b12, W_l2, W_r2, b22)

    output_head = head.reshape(_B, 1)
    target_head = targets[:, 0, :]
    return output_head, jnp.sum(diff).reshape(()), target_head
